# Initial kernel scaffold; baseline (speedup 1.0000x reference)
import jax
import jax.numpy as jnp
from jax.experimental import pallas as pl


def kernel(tokens, binary_table, analog_table, fractal_table, gate_w, gate_b):
    raise NotImplementedError("write your pallas kernel here")


# speedup vs baseline: 2.6934x; 2.6934x over previous
import functools

import jax
import jax.numpy as jnp
from jax import lax
from jax.experimental import pallas as pl
from jax.experimental.pallas import tpu as pltpu
from jax.experimental.pallas import tpu_sc as plsc

D_BIN = 32
D_ANA = 64
D_FRA = 32
D_OUT = D_BIN + D_ANA + D_FRA

_CHUNK = 128


@functools.lru_cache(maxsize=None)
def _sc_gather_fn(n_tokens: int):
    info = plsc.get_sparse_core_info()
    nc, ns = info.num_cores, info.num_subcores
    nw = nc * ns
    assert n_tokens % (nw * _CHUNK) == 0
    per_w = n_tokens // nw
    n_chunks = per_w // _CHUNK

    mesh = plsc.VectorSubcoreMesh(core_axis_name="c", subcore_axis_name="s")

    def body(tok_hbm, bin_hbm, ana_hbm, fra_hbm, braw_hbm, out_hbm,
             idx_v, bin_v, ana_v, fra_v, sem):
        wid = lax.axis_index("s") * nc + lax.axis_index("c")
        w_base = wid * per_w

        def chunk(c, carry):
            base = pl.multiple_of(w_base + c * _CHUNK, _CHUNK)
            pltpu.sync_copy(tok_hbm.at[pl.ds(base, _CHUNK)], idx_v)
            cb = pltpu.async_copy(bin_hbm.at[idx_v], bin_v, sem)
            ca = pltpu.async_copy(ana_hbm.at[idx_v], ana_v, sem)
            cf = pltpu.async_copy(fra_hbm.at[idx_v], fra_v, sem)
            cb.wait()
            ca.wait()
            cf.wait()
            pltpu.sync_copy(bin_v, braw_hbm.at[pl.ds(base, _CHUNK)])
            pltpu.sync_copy(ana_v, out_hbm.at[pl.ds(base, _CHUNK), pl.ds(D_BIN, D_ANA)])
            pltpu.sync_copy(fra_v, out_hbm.at[pl.ds(base, _CHUNK), pl.ds(D_BIN + D_ANA, D_FRA)])
            return carry

        lax.fori_loop(0, n_chunks, chunk, 0)

    return pl.kernel(
        body,
        out_type=(
            jax.ShapeDtypeStruct((n_tokens, D_BIN), jnp.float32),
            jax.ShapeDtypeStruct((n_tokens, D_OUT), jnp.float32),
        ),
        mesh=mesh,
        scratch_types=[
            pltpu.VMEM((_CHUNK,), jnp.int32),
            pltpu.VMEM((_CHUNK, D_BIN), jnp.float32),
            pltpu.VMEM((_CHUNK, D_ANA), jnp.float32),
            pltpu.VMEM((_CHUNK, D_FRA), jnp.float32),
            pltpu.SemaphoreType.DMA,
        ],
        compiler_params=pltpu.CompilerParams(use_tc_tiling_on_sc=False),
    )


def _gate_body(x_ref, w_ref, b_ref, o_ref):
    x = x_ref[...]
    y = jnp.dot(x, w_ref[...], preferred_element_type=jnp.float32)
    o_ref[...] = jnp.tanh((y + b_ref[...]) * 2.0)


def _apply_gate(b_raw, w_t, b2):
    n = b_raw.shape[0]
    tile = 2048
    assert n % tile == 0
    return pl.pallas_call(
        _gate_body,
        grid=(n // tile,),
        in_specs=[
            pl.BlockSpec((tile, D_BIN), lambda i: (i, 0)),
            pl.BlockSpec((D_BIN, D_BIN), lambda i: (0, 0)),
            pl.BlockSpec((1, D_BIN), lambda i: (0, 0)),
        ],
        out_specs=pl.BlockSpec((tile, D_BIN), lambda i: (i, 0)),
        out_shape=jax.ShapeDtypeStruct((n, D_BIN), jnp.float32),
    )(b_raw, w_t, b2)


def kernel(tokens, binary_table, analog_table, fractal_table, gate_w, gate_b):
    batch, hist = tokens.shape
    n = batch * hist
    tok = tokens.reshape(n).astype(jnp.int32)
    b_raw, out = _sc_gather_fn(n)(tok, binary_table, analog_table, fractal_table)
    g = _apply_gate(b_raw, gate_w.T, gate_b.reshape(1, D_BIN))
    out = lax.dynamic_update_slice(out, g, (0, 0))
    return out.reshape(batch, hist, D_OUT)


# speedup vs baseline: 3.1425x; 1.1667x over previous
import functools

import jax
import jax.numpy as jnp
from jax import lax
from jax.experimental import pallas as pl
from jax.experimental.pallas import tpu as pltpu
from jax.experimental.pallas import tpu_sc as plsc

D_BIN = 32
D_ANA = 64
D_FRA = 32
D_OUT = D_BIN + D_ANA + D_FRA

_CHUNK = 128
_PACK_VT = 512


def _pack_body(bt_ref, at_ref, ft_ref, o_ref):
    blk = jnp.concatenate([bt_ref[...], at_ref[...], ft_ref[...]], axis=0)
    o_ref[...] = lax.transpose(blk, (1, 0))


def _pack_tables(bt_t, at_t, ft_t):
    v = bt_t.shape[1]
    grid = (v + _PACK_VT - 1) // _PACK_VT
    return pl.pallas_call(
        _pack_body,
        grid=(grid,),
        in_specs=[
            pl.BlockSpec((D_BIN, _PACK_VT), lambda i: (0, i)),
            pl.BlockSpec((D_ANA, _PACK_VT), lambda i: (0, i)),
            pl.BlockSpec((D_FRA, _PACK_VT), lambda i: (0, i)),
        ],
        out_specs=pl.BlockSpec((_PACK_VT, D_OUT), lambda i: (i, 0)),
        out_shape=jax.ShapeDtypeStruct((v, D_OUT), jnp.float32),
    )(bt_t, at_t, ft_t)


@functools.lru_cache(maxsize=None)
def _sc_gather_fn(n_tokens: int, v: int):
    info = plsc.get_sparse_core_info()
    nc, ns = info.num_cores, info.num_subcores
    nw = nc * ns
    assert n_tokens % (nw * _CHUNK) == 0
    per_w = n_tokens // nw
    n_chunks = per_w // _CHUNK

    mesh = plsc.VectorSubcoreMesh(core_axis_name="c", subcore_axis_name="s")

    def body(tok_hbm, big_hbm, out_hbm, idx_v, rows_v, sem):
        wid = lax.axis_index("s") * nc + lax.axis_index("c")
        w_base = wid * per_w

        def chunk(c, carry):
            base = pl.multiple_of(w_base + c * _CHUNK, _CHUNK)
            pltpu.sync_copy(tok_hbm.at[pl.ds(base, _CHUNK)], idx_v)
            pltpu.async_copy(big_hbm.at[idx_v], rows_v, sem).wait()
            pltpu.sync_copy(rows_v, out_hbm.at[pl.ds(base, _CHUNK)])
            return carry

        lax.fori_loop(0, n_chunks, chunk, 0)

    return pl.kernel(
        body,
        out_type=jax.ShapeDtypeStruct((n_tokens, D_OUT), jnp.float32),
        mesh=mesh,
        scratch_types=[
            pltpu.VMEM((_CHUNK,), jnp.int32),
            pltpu.VMEM((_CHUNK, D_OUT), jnp.float32),
            pltpu.SemaphoreType.DMA,
        ],
    )


def _gate_body(x_ref, w_ref, b_ref, o_ref):
    x = x_ref[...]
    y = jnp.dot(x, w_ref[...], preferred_element_type=jnp.float32)
    o_ref[...] = jnp.tanh((y + b_ref[...]) * 2.0)


def _apply_gate(b_raw, w_t, b2):
    n = b_raw.shape[0]
    tile = 2048
    assert n % tile == 0
    return pl.pallas_call(
        _gate_body,
        grid=(n // tile,),
        in_specs=[
            pl.BlockSpec((tile, D_BIN), lambda i: (i, 0)),
            pl.BlockSpec((D_BIN, D_BIN), lambda i: (0, 0)),
            pl.BlockSpec((1, D_BIN), lambda i: (0, 0)),
        ],
        out_specs=pl.BlockSpec((tile, D_BIN), lambda i: (i, 0)),
        out_shape=jax.ShapeDtypeStruct((n, D_BIN), jnp.float32),
    )(b_raw, w_t, b2)


def kernel(tokens, binary_table, analog_table, fractal_table, gate_w, gate_b):
    batch, hist = tokens.shape
    n = batch * hist
    v = binary_table.shape[0]
    tok = tokens.T.reshape(n).astype(jnp.int32)
    big = _pack_tables(binary_table.T, analog_table.T, fractal_table.T)
    out = _sc_gather_fn(n, v)(tok, big)
    g = _apply_gate(out[:, :D_BIN], gate_w.T, gate_b.reshape(1, D_BIN))
    out = lax.dynamic_update_slice(out, g, (0, 0))
    return out.reshape(hist, batch, D_OUT).transpose(1, 0, 2)


# speedup vs baseline: 6.4805x; 2.0622x over previous
import functools

import jax
import jax.numpy as jnp
from jax import lax
from jax.experimental import pallas as pl
from jax.experimental.pallas import tpu as pltpu
from jax.experimental.pallas import tpu_sc as plsc

D_BIN = 32
D_ANA = 64
D_FRA = 32
D_OUT = D_BIN + D_ANA + D_FRA

_CHUNK = 128
_PACK_VT = 1024
_GPACK = 8


def _pack_body(bt_ref, at_ref, ft_ref, w8_ref, b8_ref, o_ref):
    c = _PACK_VT // _GPACK
    bt = bt_ref[...]
    rhs = jnp.concatenate(
        [bt[:, m * c:(m + 1) * c] for m in range(_GPACK)], axis=0)
    y8 = jnp.dot(w8_ref[...], rhs, preferred_element_type=jnp.float32)
    g8 = jnp.tanh((y8 + b8_ref[...]) * 2.0)
    gated = jnp.concatenate(
        [g8[m * D_BIN:(m + 1) * D_BIN, :] for m in range(_GPACK)], axis=1)
    blk = jnp.concatenate([gated, at_ref[...], ft_ref[...]], axis=0)
    o_ref[...] = lax.transpose(blk, (1, 0))


def _pack_tables(bt_t, at_t, ft_t, w8, b8):
    v = bt_t.shape[1]
    grid = (v + _PACK_VT - 1) // _PACK_VT
    return pl.pallas_call(
        _pack_body,
        grid=(grid,),
        in_specs=[
            pl.BlockSpec((D_BIN, _PACK_VT), lambda i: (0, i)),
            pl.BlockSpec((D_ANA, _PACK_VT), lambda i: (0, i)),
            pl.BlockSpec((D_FRA, _PACK_VT), lambda i: (0, i)),
            pl.BlockSpec((_GPACK * D_BIN, _GPACK * D_BIN), lambda i: (0, 0)),
            pl.BlockSpec((_GPACK * D_BIN, 1), lambda i: (0, 0)),
        ],
        out_specs=pl.BlockSpec((_PACK_VT, D_OUT), lambda i: (i, 0)),
        out_shape=jax.ShapeDtypeStruct((v, D_OUT), jnp.float32),
    )(bt_t, at_t, ft_t, w8, b8)


@functools.lru_cache(maxsize=None)
def _sc_gather_fn(n_tokens: int, v: int):
    info = plsc.get_sparse_core_info()
    nc, ns = info.num_cores, info.num_subcores
    nw = nc * ns
    assert n_tokens % (nw * _CHUNK) == 0
    per_w = n_tokens // nw
    n_chunks = per_w // _CHUNK

    mesh = plsc.VectorSubcoreMesh(core_axis_name="c", subcore_axis_name="s")

    def body(tok_hbm, big_hbm, out_hbm, idx_v, rows_v, sem):
        wid = lax.axis_index("s") * nc + lax.axis_index("c")
        w_base = wid * per_w

        def chunk(c, carry):
            base = pl.multiple_of(w_base + c * _CHUNK, _CHUNK)
            pltpu.sync_copy(tok_hbm.at[pl.ds(base, _CHUNK)], idx_v)
            pltpu.async_copy(big_hbm.at[idx_v], rows_v, sem).wait()
            pltpu.sync_copy(rows_v, out_hbm.at[pl.ds(base, _CHUNK)])
            return carry

        lax.fori_loop(0, n_chunks, chunk, 0)

    return pl.kernel(
        body,
        out_type=jax.ShapeDtypeStruct((n_tokens, D_OUT), jnp.float32),
        mesh=mesh,
        scratch_types=[
            pltpu.VMEM((_CHUNK,), jnp.int32),
            pltpu.VMEM((_CHUNK, D_OUT), jnp.float32),
            pltpu.SemaphoreType.DMA,
        ],
    )


def _gate_mats(gate_w, gate_b):
    eye = jnp.eye(_GPACK, dtype=jnp.float32)
    w8 = jnp.einsum("ml,ji->mjli", eye, gate_w).reshape(
        _GPACK * D_BIN, _GPACK * D_BIN)
    b8 = jnp.tile(gate_b, _GPACK).reshape(_GPACK * D_BIN, 1)
    return w8, b8


def kernel(tokens, binary_table, analog_table, fractal_table, gate_w, gate_b):
    batch, hist = tokens.shape
    n = batch * hist
    v = binary_table.shape[0]
    tok = tokens.T.reshape(n).astype(jnp.int32)
    w8, b8 = _gate_mats(gate_w, gate_b)
    big = _pack_tables(binary_table.T, analog_table.T, fractal_table.T, w8, b8)
    out = _sc_gather_fn(n, v)(tok, big)
    return out.reshape(hist, batch, D_OUT).transpose(1, 0, 2)


# speedup vs baseline: 11.4246x; 1.7629x over previous
import functools

import jax
import jax.numpy as jnp
from jax import lax
from jax.experimental import pallas as pl
from jax.experimental.pallas import tpu as pltpu
from jax.experimental.pallas import tpu_sc as plsc

D_BIN = 32
D_ANA = 64
D_FRA = 32
D_OUT = D_BIN + D_ANA + D_FRA

_CHUNK = 128
_PACK_VT = 4096
_GPACK = 8


def _pack_body(bt_ref, at_ref, ft_ref, w8_ref, b8_ref, o_ref):
    c = _PACK_VT // _GPACK
    bt = bt_ref[...]
    rhs = jnp.concatenate(
        [bt[:, m * c:(m + 1) * c] for m in range(_GPACK)], axis=0)
    y8 = jnp.dot(w8_ref[...], rhs, preferred_element_type=jnp.float32)
    g8 = jnp.tanh((y8 + b8_ref[...]) * 2.0)
    gated = jnp.concatenate(
        [g8[m * D_BIN:(m + 1) * D_BIN, :] for m in range(_GPACK)], axis=1)
    blk = jnp.concatenate([gated, at_ref[...], ft_ref[...]], axis=0)
    o_ref[...] = lax.transpose(blk, (1, 0))


def _pack_tables(bt_t, at_t, ft_t, w8, b8):
    v = bt_t.shape[1]
    grid = (v + _PACK_VT - 1) // _PACK_VT
    return pl.pallas_call(
        _pack_body,
        grid=(grid,),
        in_specs=[
            pl.BlockSpec((D_BIN, _PACK_VT), lambda i: (0, i)),
            pl.BlockSpec((D_ANA, _PACK_VT), lambda i: (0, i)),
            pl.BlockSpec((D_FRA, _PACK_VT), lambda i: (0, i)),
            pl.BlockSpec((_GPACK * D_BIN, _GPACK * D_BIN), lambda i: (0, 0)),
            pl.BlockSpec((_GPACK * D_BIN, 1), lambda i: (0, 0)),
        ],
        out_specs=pl.BlockSpec((_PACK_VT, D_OUT), lambda i: (i, 0)),
        out_shape=jax.ShapeDtypeStruct((v, D_OUT), jnp.float32),
    )(bt_t, at_t, ft_t, w8, b8)


@functools.lru_cache(maxsize=None)
def _sc_gather_fn(n_tokens: int, v: int):
    info = plsc.get_sparse_core_info()
    nc, ns = info.num_cores, info.num_subcores
    nw = nc * ns
    assert n_tokens % (nw * _CHUNK) == 0
    per_w = n_tokens // nw
    n_chunks = per_w // _CHUNK

    mesh = plsc.VectorSubcoreMesh(core_axis_name="c", subcore_axis_name="s")

    def body(tok_hbm, big_hbm, out_hbm, idx_v, rows_v, sem):
        wid = lax.axis_index("s") * nc + lax.axis_index("c")
        w_base = wid * per_w

        def chunk(c, carry):
            base = pl.multiple_of(w_base + c * _CHUNK, _CHUNK)
            pltpu.sync_copy(tok_hbm.at[pl.ds(base, _CHUNK)], idx_v)
            pltpu.async_copy(big_hbm.at[idx_v], rows_v, sem).wait()
            pltpu.sync_copy(rows_v, out_hbm.at[pl.ds(base, _CHUNK)])
            return carry

        lax.fori_loop(0, n_chunks, chunk, 0)

    return pl.kernel(
        body,
        out_type=jax.ShapeDtypeStruct((n_tokens, D_OUT), jnp.float32),
        mesh=mesh,
        scratch_types=[
            pltpu.VMEM((_CHUNK,), jnp.int32),
            pltpu.VMEM((_CHUNK, D_OUT), jnp.float32),
            pltpu.SemaphoreType.DMA,
        ],
    )


def _gate_mats(gate_w, gate_b):
    eye = jnp.eye(_GPACK, dtype=jnp.float32)
    w8 = jnp.einsum("ml,ji->mjli", eye, gate_w).reshape(
        _GPACK * D_BIN, _GPACK * D_BIN)
    b8 = jnp.tile(gate_b, _GPACK).reshape(_GPACK * D_BIN, 1)
    return w8, b8


def kernel(tokens, binary_table, analog_table, fractal_table, gate_w, gate_b):
    batch, hist = tokens.shape
    n = batch * hist
    v = binary_table.shape[0]
    tok = tokens.T.reshape(n).astype(jnp.int32)
    w8, b8 = _gate_mats(gate_w, gate_b)
    big = _pack_tables(binary_table.T, analog_table.T, fractal_table.T, w8, b8)
    out = _sc_gather_fn(n, v)(tok, big)
    return out.reshape(hist, batch, D_OUT).transpose(1, 0, 2)


# speedup vs baseline: 13.2021x; 1.1556x over previous
import functools

import jax
import jax.numpy as jnp
from jax import lax
from jax.experimental import pallas as pl
from jax.experimental.pallas import tpu as pltpu
from jax.experimental.pallas import tpu_sc as plsc

D_BIN = 32
D_ANA = 64
D_FRA = 32
D_OUT = D_BIN + D_ANA + D_FRA

_CHUNK = 128
_PACK_VT = 8192
_GPACK = 8


def _pack_body(bt_ref, at_ref, ft_ref, w8_ref, b8_ref, o_ref):
    c = _PACK_VT // _GPACK
    bt = bt_ref[...]
    rhs = jnp.concatenate(
        [bt[:, m * c:(m + 1) * c] for m in range(_GPACK)], axis=0)
    y8 = jnp.dot(w8_ref[...], rhs, preferred_element_type=jnp.float32)
    g8 = jnp.tanh((y8 + b8_ref[...]) * 2.0)
    gated = jnp.concatenate(
        [g8[m * D_BIN:(m + 1) * D_BIN, :] for m in range(_GPACK)], axis=1)
    blk = jnp.concatenate([gated, at_ref[...], ft_ref[...]], axis=0)
    o_ref[...] = lax.transpose(blk, (1, 0))


def _pack_tables(bt_t, at_t, ft_t, w8, b8):
    v = bt_t.shape[1]
    grid = (v + _PACK_VT - 1) // _PACK_VT
    return pl.pallas_call(
        _pack_body,
        grid=(grid,),
        in_specs=[
            pl.BlockSpec((D_BIN, _PACK_VT), lambda i: (0, i)),
            pl.BlockSpec((D_ANA, _PACK_VT), lambda i: (0, i)),
            pl.BlockSpec((D_FRA, _PACK_VT), lambda i: (0, i)),
            pl.BlockSpec((_GPACK * D_BIN, _GPACK * D_BIN), lambda i: (0, 0)),
            pl.BlockSpec((_GPACK * D_BIN, 1), lambda i: (0, 0)),
        ],
        out_specs=pl.BlockSpec((_PACK_VT, D_OUT), lambda i: (i, 0)),
        out_shape=jax.ShapeDtypeStruct((v, D_OUT), jnp.float32),
    )(bt_t, at_t, ft_t, w8, b8)


@functools.lru_cache(maxsize=None)
def _sc_gather_fn(n_tokens: int, v: int):
    info = plsc.get_sparse_core_info()
    nc, ns = info.num_cores, info.num_subcores
    nw = nc * ns
    assert n_tokens % (nw * _CHUNK) == 0
    per_w = n_tokens // nw
    n_chunks = per_w // _CHUNK

    mesh = plsc.VectorSubcoreMesh(core_axis_name="c", subcore_axis_name="s")

    def body(tok_hbm, big_hbm, out_hbm, idx_v, rows_v, sem):
        wid = lax.axis_index("s") * nc + lax.axis_index("c")
        w_base = wid * per_w

        def chunk(c, carry):
            base = pl.multiple_of(w_base + c * _CHUNK, _CHUNK)
            pltpu.sync_copy(tok_hbm.at[pl.ds(base, _CHUNK)], idx_v)
            pltpu.async_copy(big_hbm.at[idx_v], rows_v, sem).wait()
            pltpu.sync_copy(rows_v, out_hbm.at[pl.ds(base, _CHUNK)])
            return carry

        lax.fori_loop(0, n_chunks, chunk, 0)

    return pl.kernel(
        body,
        out_type=jax.ShapeDtypeStruct((n_tokens, D_OUT), jnp.float32),
        mesh=mesh,
        scratch_types=[
            pltpu.VMEM((_CHUNK,), jnp.int32),
            pltpu.VMEM((_CHUNK, D_OUT), jnp.float32),
            pltpu.SemaphoreType.DMA,
        ],
    )


def _gate_mats(gate_w, gate_b):
    eye = jnp.eye(_GPACK, dtype=jnp.float32)
    w8 = jnp.einsum("ml,ji->mjli", eye, gate_w).reshape(
        _GPACK * D_BIN, _GPACK * D_BIN)
    b8 = jnp.tile(gate_b, _GPACK).reshape(_GPACK * D_BIN, 1)
    return w8, b8


def kernel(tokens, binary_table, analog_table, fractal_table, gate_w, gate_b):
    batch, hist = tokens.shape
    n = batch * hist
    v = binary_table.shape[0]
    tok = tokens.T.reshape(n).astype(jnp.int32)
    w8, b8 = _gate_mats(gate_w, gate_b)
    big = _pack_tables(binary_table.T, analog_table.T, fractal_table.T, w8, b8)
    out = _sc_gather_fn(n, v)(tok, big)
    return out.reshape(hist, batch, D_OUT).transpose(1, 0, 2)


# speedup vs baseline: 13.5506x; 1.0264x over previous
import functools

import jax
import jax.numpy as jnp
from jax import lax
from jax.experimental import pallas as pl
from jax.experimental.pallas import tpu as pltpu
from jax.experimental.pallas import tpu_sc as plsc

D_BIN = 32
D_ANA = 64
D_FRA = 32
D_OUT = D_BIN + D_ANA + D_FRA

_CHUNK = 128
_PACK_VT = 16384
_GPACK = 8


def _pack_body(bt_ref, at_ref, ft_ref, w8_ref, b8_ref, o_ref):
    c = _PACK_VT // _GPACK
    bt = bt_ref[...]
    rhs = jnp.concatenate(
        [bt[:, m * c:(m + 1) * c] for m in range(_GPACK)], axis=0)
    y8 = jnp.dot(w8_ref[...], rhs, preferred_element_type=jnp.float32)
    g8 = jnp.tanh((y8 + b8_ref[...]) * 2.0)
    gated = jnp.concatenate(
        [g8[m * D_BIN:(m + 1) * D_BIN, :] for m in range(_GPACK)], axis=1)
    blk = jnp.concatenate([gated, at_ref[...], ft_ref[...]], axis=0)
    o_ref[...] = lax.transpose(blk, (1, 0))


def _pack_tables(bt_t, at_t, ft_t, w8, b8):
    v = bt_t.shape[1]
    grid = (v + _PACK_VT - 1) // _PACK_VT
    return pl.pallas_call(
        _pack_body,
        grid=(grid,),
        in_specs=[
            pl.BlockSpec((D_BIN, _PACK_VT), lambda i: (0, i)),
            pl.BlockSpec((D_ANA, _PACK_VT), lambda i: (0, i)),
            pl.BlockSpec((D_FRA, _PACK_VT), lambda i: (0, i)),
            pl.BlockSpec((_GPACK * D_BIN, _GPACK * D_BIN), lambda i: (0, 0)),
            pl.BlockSpec((_GPACK * D_BIN, 1), lambda i: (0, 0)),
        ],
        out_specs=pl.BlockSpec((_PACK_VT, D_OUT), lambda i: (i, 0)),
        out_shape=jax.ShapeDtypeStruct((v, D_OUT), jnp.float32),
    )(bt_t, at_t, ft_t, w8, b8)


@functools.lru_cache(maxsize=None)
def _sc_gather_fn(n_tokens: int, v: int):
    info = plsc.get_sparse_core_info()
    nc, ns = info.num_cores, info.num_subcores
    nw = nc * ns
    assert n_tokens % (nw * _CHUNK) == 0
    per_w = n_tokens // nw
    n_chunks = per_w // _CHUNK

    mesh = plsc.VectorSubcoreMesh(core_axis_name="c", subcore_axis_name="s")

    def body(tok_hbm, big_hbm, out_hbm, idx_v, rows_v, sem):
        wid = lax.axis_index("s") * nc + lax.axis_index("c")
        w_base = wid * per_w

        def chunk(c, carry):
            base = pl.multiple_of(w_base + c * _CHUNK, _CHUNK)
            pltpu.sync_copy(tok_hbm.at[pl.ds(base, _CHUNK)], idx_v)
            pltpu.async_copy(big_hbm.at[idx_v], rows_v, sem).wait()
            pltpu.sync_copy(rows_v, out_hbm.at[pl.ds(base, _CHUNK)])
            return carry

        lax.fori_loop(0, n_chunks, chunk, 0)

    return pl.kernel(
        body,
        out_type=jax.ShapeDtypeStruct((n_tokens, D_OUT), jnp.float32),
        mesh=mesh,
        scratch_types=[
            pltpu.VMEM((_CHUNK,), jnp.int32),
            pltpu.VMEM((_CHUNK, D_OUT), jnp.float32),
            pltpu.SemaphoreType.DMA,
        ],
    )


def _gate_mats(gate_w, gate_b):
    eye = jnp.eye(_GPACK, dtype=jnp.float32)
    w8 = jnp.einsum("ml,ji->mjli", eye, gate_w).reshape(
        _GPACK * D_BIN, _GPACK * D_BIN)
    b8 = jnp.tile(gate_b, _GPACK).reshape(_GPACK * D_BIN, 1)
    return w8, b8


def kernel(tokens, binary_table, analog_table, fractal_table, gate_w, gate_b):
    batch, hist = tokens.shape
    n = batch * hist
    v = binary_table.shape[0]
    tok = tokens.T.reshape(n).astype(jnp.int32)
    w8, b8 = _gate_mats(gate_w, gate_b)
    big = _pack_tables(binary_table.T, analog_table.T, fractal_table.T, w8, b8)
    out = _sc_gather_fn(n, v)(tok, big)
    return out.reshape(hist, batch, D_OUT).transpose(1, 0, 2)


# speedup vs baseline: 15.4318x; 1.1388x over previous
import functools

import jax
import jax.numpy as jnp
from jax import lax
from jax.experimental import pallas as pl
from jax.experimental.pallas import tpu as pltpu
from jax.experimental.pallas import tpu_sc as plsc

D_BIN = 32
D_ANA = 64
D_FRA = 32
D_OUT = D_BIN + D_ANA + D_FRA

_CHUNK = 128
_PACK_VT = 16384
_GPACK = 8


def _pack_body(bt_ref, at_ref, ft_ref, w8_ref, b8_ref, o_ref):
    c = _PACK_VT // _GPACK
    bt = bt_ref[...]
    rhs = jnp.concatenate(
        [bt[:, m * c:(m + 1) * c] for m in range(_GPACK)], axis=0)
    y8 = jnp.dot(w8_ref[...], rhs, preferred_element_type=jnp.float32)
    g8 = jnp.tanh((y8 + b8_ref[...]) * 2.0)
    gated = jnp.concatenate(
        [g8[m * D_BIN:(m + 1) * D_BIN, :] for m in range(_GPACK)], axis=1)
    blk = jnp.concatenate([gated, at_ref[...], ft_ref[...]], axis=0)
    o_ref[...] = lax.transpose(blk, (1, 0))


def _pack_tables(bt_t, at_t, ft_t, w8, b8):
    v = bt_t.shape[1]
    grid = (v + _PACK_VT - 1) // _PACK_VT
    return pl.pallas_call(
        _pack_body,
        grid=(grid,),
        in_specs=[
            pl.BlockSpec((D_BIN, _PACK_VT), lambda i: (0, i)),
            pl.BlockSpec((D_ANA, _PACK_VT), lambda i: (0, i)),
            pl.BlockSpec((D_FRA, _PACK_VT), lambda i: (0, i)),
            pl.BlockSpec((_GPACK * D_BIN, _GPACK * D_BIN), lambda i: (0, 0)),
            pl.BlockSpec((_GPACK * D_BIN, 1), lambda i: (0, 0)),
        ],
        out_specs=pl.BlockSpec((_PACK_VT, D_OUT), lambda i: (i, 0)),
        out_shape=jax.ShapeDtypeStruct((v, D_OUT), jnp.float32),
    )(bt_t, at_t, ft_t, w8, b8)


_NBUF = 5


@functools.lru_cache(maxsize=None)
def _sc_gather_fn(n_tokens: int, v: int):
    info = plsc.get_sparse_core_info()
    nc, ns = info.num_cores, info.num_subcores
    nw = nc * ns
    assert n_tokens % (nw * _CHUNK * _NBUF) == 0
    per_w = n_tokens // nw
    n_groups = per_w // (_CHUNK * _NBUF)

    mesh = plsc.VectorSubcoreMesh(core_axis_name="c", subcore_axis_name="s")

    def body(tok_hbm, big_hbm, out_hbm, idx_v, *bufs_and_sems):
        rows = bufs_and_sems[:_NBUF]
        sem_g = bufs_and_sems[_NBUF:2 * _NBUF]
        sem_w = bufs_and_sems[2 * _NBUF:3 * _NBUF]
        wid = lax.axis_index("s") * nc + lax.axis_index("c")
        w_base = wid * per_w
        pltpu.sync_copy(tok_hbm.at[pl.ds(w_base, per_w)], idx_v)

        def out_slice(g, b):
            base = pl.multiple_of(
                w_base + (g * _NBUF + b) * _CHUNK, _CHUNK)
            return out_hbm.at[pl.ds(base, _CHUNK)]

        def idx_slice(g, b):
            off = pl.multiple_of((g * _NBUF + b) * _CHUNK, _CHUNK)
            return idx_v.at[pl.ds(off, _CHUNK)]

        def group(g, carry):
            gathers = []
            for b in range(_NBUF):
                @pl.when(g > 0)
                def _(b=b):
                    pltpu.make_async_copy(
                        rows[b], out_slice(g - 1, b), sem_w[b]).wait()
                gathers.append(pltpu.async_copy(
                    big_hbm.at[idx_slice(g, b)], rows[b], sem_g[b]))
            for b in range(_NBUF):
                gathers[b].wait()
                pltpu.async_copy(rows[b], out_slice(g, b), sem_w[b])
            return carry

        lax.fori_loop(0, n_groups, group, 0)
        for b in range(_NBUF):
            pltpu.make_async_copy(
                rows[b], out_slice(n_groups - 1, b), sem_w[b]).wait()

    return pl.kernel(
        body,
        out_type=jax.ShapeDtypeStruct((n_tokens, D_OUT), jnp.float32),
        mesh=mesh,
        scratch_types=(
            [pltpu.VMEM((n_tokens // nw,), jnp.int32)]
            + [pltpu.VMEM((_CHUNK, D_OUT), jnp.float32)] * _NBUF
            + [pltpu.SemaphoreType.DMA] * (2 * _NBUF)
        ),
    )


def _gate_mats(gate_w, gate_b):
    eye = jnp.eye(_GPACK, dtype=jnp.float32)
    w8 = jnp.einsum("ml,ji->mjli", eye, gate_w).reshape(
        _GPACK * D_BIN, _GPACK * D_BIN)
    b8 = jnp.tile(gate_b, _GPACK).reshape(_GPACK * D_BIN, 1)
    return w8, b8


def kernel(tokens, binary_table, analog_table, fractal_table, gate_w, gate_b):
    batch, hist = tokens.shape
    n = batch * hist
    v = binary_table.shape[0]
    tok = tokens.T.reshape(n).astype(jnp.int32)
    w8, b8 = _gate_mats(gate_w, gate_b)
    big = _pack_tables(binary_table.T, analog_table.T, fractal_table.T, w8, b8)
    out = _sc_gather_fn(n, v)(tok, big)
    return out.reshape(hist, batch, D_OUT).transpose(1, 0, 2)


# speedup vs baseline: 15.4337x; 1.0001x over previous
import functools

import jax
import jax.numpy as jnp
from jax import lax
from jax.experimental import pallas as pl
from jax.experimental.pallas import tpu as pltpu
from jax.experimental.pallas import tpu_sc as plsc

D_BIN = 32
D_ANA = 64
D_FRA = 32
D_OUT = D_BIN + D_ANA + D_FRA

_CHUNK = 128
_PACK_VT = 16384
_GPACK = 8


def _pack_body(bt_ref, at_ref, ft_ref, w8_ref, b8_ref, o_ref):
    c = _PACK_VT // _GPACK
    bt = bt_ref[...]
    rhs = jnp.concatenate(
        [bt[:, m * c:(m + 1) * c] for m in range(_GPACK)], axis=0)
    y8 = jnp.dot(w8_ref[...], rhs, preferred_element_type=jnp.float32)
    g8 = jnp.tanh((y8 + b8_ref[...]) * 2.0)
    gated = jnp.concatenate(
        [g8[m * D_BIN:(m + 1) * D_BIN, :] for m in range(_GPACK)], axis=1)
    blk = jnp.concatenate([gated, at_ref[...], ft_ref[...]], axis=0)
    o_ref[...] = lax.transpose(blk, (1, 0))


def _pack_tables(bt_t, at_t, ft_t, w8, b8):
    v = bt_t.shape[1]
    grid = (v + _PACK_VT - 1) // _PACK_VT
    return pl.pallas_call(
        _pack_body,
        grid=(grid,),
        in_specs=[
            pl.BlockSpec((D_BIN, _PACK_VT), lambda i: (0, i)),
            pl.BlockSpec((D_ANA, _PACK_VT), lambda i: (0, i)),
            pl.BlockSpec((D_FRA, _PACK_VT), lambda i: (0, i)),
            pl.BlockSpec((_GPACK * D_BIN, _GPACK * D_BIN), lambda i: (0, 0)),
            pl.BlockSpec((_GPACK * D_BIN, 1), lambda i: (0, 0)),
        ],
        out_specs=pl.BlockSpec((_PACK_VT, D_OUT), lambda i: (i, 0)),
        out_shape=jax.ShapeDtypeStruct((v, D_OUT), jnp.float32),
        compiler_params=pltpu.CompilerParams(
            vmem_limit_bytes=100 * 1024 * 1024),
    )(bt_t, at_t, ft_t, w8, b8)


_NBUF = 5


@functools.lru_cache(maxsize=None)
def _sc_gather_fn(n_tokens: int, v: int):
    info = plsc.get_sparse_core_info()
    nc, ns = info.num_cores, info.num_subcores
    nw = nc * ns
    assert n_tokens % (nw * _CHUNK * _NBUF) == 0
    per_w = n_tokens // nw
    n_groups = per_w // (_CHUNK * _NBUF)

    mesh = plsc.VectorSubcoreMesh(core_axis_name="c", subcore_axis_name="s")

    def body(tok_hbm, big_hbm, out_hbm, idx_v, *bufs_and_sems):
        rows = bufs_and_sems[:_NBUF]
        sem_g = bufs_and_sems[_NBUF:2 * _NBUF]
        sem_w = bufs_and_sems[2 * _NBUF:3 * _NBUF]
        wid = lax.axis_index("s") * nc + lax.axis_index("c")
        w_base = wid * per_w
        pltpu.sync_copy(tok_hbm.at[pl.ds(w_base, per_w)], idx_v)

        def out_slice(g, b):
            base = pl.multiple_of(
                w_base + (g * _NBUF + b) * _CHUNK, _CHUNK)
            return out_hbm.at[pl.ds(base, _CHUNK)]

        def idx_slice(g, b):
            off = pl.multiple_of((g * _NBUF + b) * _CHUNK, _CHUNK)
            return idx_v.at[pl.ds(off, _CHUNK)]

        def group(g, carry):
            gathers = []
            for b in range(_NBUF):
                @pl.when(g > 0)
                def _(b=b):
                    pltpu.make_async_copy(
                        rows[b], out_slice(g - 1, b), sem_w[b]).wait()
                gathers.append(pltpu.async_copy(
                    big_hbm.at[idx_slice(g, b)], rows[b], sem_g[b]))
            for b in range(_NBUF):
                gathers[b].wait()
                pltpu.async_copy(rows[b], out_slice(g, b), sem_w[b])
            return carry

        lax.fori_loop(0, n_groups, group, 0)
        for b in range(_NBUF):
            pltpu.make_async_copy(
                rows[b], out_slice(n_groups - 1, b), sem_w[b]).wait()

    return pl.kernel(
        body,
        out_type=jax.ShapeDtypeStruct((n_tokens, D_OUT), jnp.float32),
        mesh=mesh,
        scratch_types=(
            [pltpu.VMEM((n_tokens // nw,), jnp.int32)]
            + [pltpu.VMEM((_CHUNK, D_OUT), jnp.float32)] * _NBUF
            + [pltpu.SemaphoreType.DMA] * (2 * _NBUF)
        ),
    )


def _gate_mats(gate_w, gate_b):
    eye = jnp.eye(_GPACK, dtype=jnp.float32)
    w8 = jnp.einsum("ml,ji->mjli", eye, gate_w).reshape(
        _GPACK * D_BIN, _GPACK * D_BIN)
    b8 = jnp.tile(gate_b, _GPACK).reshape(_GPACK * D_BIN, 1)
    return w8, b8


def kernel(tokens, binary_table, analog_table, fractal_table, gate_w, gate_b):
    batch, hist = tokens.shape
    n = batch * hist
    v = binary_table.shape[0]
    tok = tokens.T.reshape(n).astype(jnp.int32)
    w8, b8 = _gate_mats(gate_w, gate_b)
    big = _pack_tables(binary_table.T, analog_table.T, fractal_table.T, w8, b8)
    out = _sc_gather_fn(n, v)(tok, big)
    return out.reshape(hist, batch, D_OUT).transpose(1, 0, 2)


# speedup vs baseline: 15.4952x; 1.0040x over previous
import functools

import jax
import jax.numpy as jnp
from jax import lax
from jax.experimental import pallas as pl
from jax.experimental.pallas import tpu as pltpu
from jax.experimental.pallas import tpu_sc as plsc

D_BIN = 32
D_ANA = 64
D_FRA = 32
D_OUT = D_BIN + D_ANA + D_FRA

_CHUNK = 64
_PACK_VT = 16384
_GPACK = 8


def _pack_body(bt_ref, at_ref, ft_ref, w8_ref, b8_ref, o_ref):
    c = _PACK_VT // _GPACK
    bt = bt_ref[...]
    rhs = jnp.concatenate(
        [bt[:, m * c:(m + 1) * c] for m in range(_GPACK)], axis=0)
    y8 = jnp.dot(w8_ref[...], rhs, preferred_element_type=jnp.float32)
    g8 = jnp.tanh((y8 + b8_ref[...]) * 2.0)
    gated = jnp.concatenate(
        [g8[m * D_BIN:(m + 1) * D_BIN, :] for m in range(_GPACK)], axis=1)
    blk = jnp.concatenate([gated, at_ref[...], ft_ref[...]], axis=0)
    o_ref[...] = lax.transpose(blk, (1, 0))


def _pack_tables(bt_t, at_t, ft_t, w8, b8):
    v = bt_t.shape[1]
    grid = (v + _PACK_VT - 1) // _PACK_VT
    return pl.pallas_call(
        _pack_body,
        grid=(grid,),
        in_specs=[
            pl.BlockSpec((D_BIN, _PACK_VT), lambda i: (0, i)),
            pl.BlockSpec((D_ANA, _PACK_VT), lambda i: (0, i)),
            pl.BlockSpec((D_FRA, _PACK_VT), lambda i: (0, i)),
            pl.BlockSpec((_GPACK * D_BIN, _GPACK * D_BIN), lambda i: (0, 0)),
            pl.BlockSpec((_GPACK * D_BIN, 1), lambda i: (0, 0)),
        ],
        out_specs=pl.BlockSpec((_PACK_VT, D_OUT), lambda i: (i, 0)),
        out_shape=jax.ShapeDtypeStruct((v, D_OUT), jnp.float32),
        compiler_params=pltpu.CompilerParams(
            vmem_limit_bytes=100 * 1024 * 1024),
    )(bt_t, at_t, ft_t, w8, b8)


_NBUF = 10


@functools.lru_cache(maxsize=None)
def _sc_gather_fn(n_tokens: int, v: int):
    info = plsc.get_sparse_core_info()
    nc, ns = info.num_cores, info.num_subcores
    nw = nc * ns
    assert n_tokens % (nw * _CHUNK * _NBUF) == 0
    per_w = n_tokens // nw
    n_groups = per_w // (_CHUNK * _NBUF)

    mesh = plsc.VectorSubcoreMesh(core_axis_name="c", subcore_axis_name="s")

    def body(tok_hbm, big_hbm, out_hbm, idx_v, *bufs_and_sems):
        rows = bufs_and_sems[:_NBUF]
        sem_g = bufs_and_sems[_NBUF:2 * _NBUF]
        sem_w = bufs_and_sems[2 * _NBUF:3 * _NBUF]
        wid = lax.axis_index("s") * nc + lax.axis_index("c")
        w_base = wid * per_w
        pltpu.sync_copy(tok_hbm.at[pl.ds(w_base, per_w)], idx_v)

        def out_slice(g, b):
            base = pl.multiple_of(
                w_base + (g * _NBUF + b) * _CHUNK, _CHUNK)
            return out_hbm.at[pl.ds(base, _CHUNK)]

        def idx_slice(g, b):
            off = pl.multiple_of((g * _NBUF + b) * _CHUNK, _CHUNK)
            return idx_v.at[pl.ds(off, _CHUNK)]

        def group(g, carry):
            gathers = []
            for b in range(_NBUF):
                @pl.when(g > 0)
                def _(b=b):
                    pltpu.make_async_copy(
                        rows[b], out_slice(g - 1, b), sem_w[b]).wait()
                gathers.append(pltpu.async_copy(
                    big_hbm.at[idx_slice(g, b)], rows[b], sem_g[b]))
            for b in range(_NBUF):
                gathers[b].wait()
                pltpu.async_copy(rows[b], out_slice(g, b), sem_w[b])
            return carry

        lax.fori_loop(0, n_groups, group, 0)
        for b in range(_NBUF):
            pltpu.make_async_copy(
                rows[b], out_slice(n_groups - 1, b), sem_w[b]).wait()

    return pl.kernel(
        body,
        out_type=jax.ShapeDtypeStruct((n_tokens, D_OUT), jnp.float32),
        mesh=mesh,
        scratch_types=(
            [pltpu.VMEM((n_tokens // nw,), jnp.int32)]
            + [pltpu.VMEM((_CHUNK, D_OUT), jnp.float32)] * _NBUF
            + [pltpu.SemaphoreType.DMA] * (2 * _NBUF)
        ),
    )


def _gate_mats(gate_w, gate_b):
    eye = jnp.eye(_GPACK, dtype=jnp.float32)
    w8 = jnp.einsum("ml,ji->mjli", eye, gate_w).reshape(
        _GPACK * D_BIN, _GPACK * D_BIN)
    b8 = jnp.tile(gate_b, _GPACK).reshape(_GPACK * D_BIN, 1)
    return w8, b8


def kernel(tokens, binary_table, analog_table, fractal_table, gate_w, gate_b):
    batch, hist = tokens.shape
    n = batch * hist
    v = binary_table.shape[0]
    tok = tokens.T.reshape(n).astype(jnp.int32)
    w8, b8 = _gate_mats(gate_w, gate_b)
    big = _pack_tables(binary_table.T, analog_table.T, fractal_table.T, w8, b8)
    out = _sc_gather_fn(n, v)(tok, big)
    return out.reshape(hist, batch, D_OUT).transpose(1, 0, 2)


# speedup vs baseline: 15.5518x; 1.0037x over previous
import functools

import jax
import jax.numpy as jnp
from jax import lax
from jax.experimental import pallas as pl
from jax.experimental.pallas import tpu as pltpu
from jax.experimental.pallas import tpu_sc as plsc

D_BIN = 32
D_ANA = 64
D_FRA = 32
D_OUT = D_BIN + D_ANA + D_FRA

_CHUNK = 64
_PACK_VT = 24576
_GPACK = 8


def _pack_body(bt_ref, at_ref, ft_ref, w8_ref, b8_ref, o_ref):
    c = _PACK_VT // _GPACK
    bt = bt_ref[...]
    rhs = jnp.concatenate(
        [bt[:, m * c:(m + 1) * c] for m in range(_GPACK)], axis=0)
    y8 = jnp.dot(w8_ref[...], rhs, preferred_element_type=jnp.float32)
    g8 = jnp.tanh((y8 + b8_ref[...]) * 2.0)
    gated = jnp.concatenate(
        [g8[m * D_BIN:(m + 1) * D_BIN, :] for m in range(_GPACK)], axis=1)
    blk = jnp.concatenate([gated, at_ref[...], ft_ref[...]], axis=0)
    o_ref[...] = lax.transpose(blk, (1, 0))


def _pack_tables(bt_t, at_t, ft_t, w8, b8):
    v = bt_t.shape[1]
    grid = (v + _PACK_VT - 1) // _PACK_VT
    return pl.pallas_call(
        _pack_body,
        grid=(grid,),
        in_specs=[
            pl.BlockSpec((D_BIN, _PACK_VT), lambda i: (0, i)),
            pl.BlockSpec((D_ANA, _PACK_VT), lambda i: (0, i)),
            pl.BlockSpec((D_FRA, _PACK_VT), lambda i: (0, i)),
            pl.BlockSpec((_GPACK * D_BIN, _GPACK * D_BIN), lambda i: (0, 0)),
            pl.BlockSpec((_GPACK * D_BIN, 1), lambda i: (0, 0)),
        ],
        out_specs=pl.BlockSpec((_PACK_VT, D_OUT), lambda i: (i, 0)),
        out_shape=jax.ShapeDtypeStruct((v, D_OUT), jnp.float32),
        compiler_params=pltpu.CompilerParams(
            vmem_limit_bytes=100 * 1024 * 1024),
    )(bt_t, at_t, ft_t, w8, b8)


_NBUF = 10


@functools.lru_cache(maxsize=None)
def _sc_gather_fn(n_tokens: int, v: int):
    info = plsc.get_sparse_core_info()
    nc, ns = info.num_cores, info.num_subcores
    nw = nc * ns
    assert n_tokens % (nw * _CHUNK * _NBUF) == 0
    per_w = n_tokens // nw
    n_groups = per_w // (_CHUNK * _NBUF)

    mesh = plsc.VectorSubcoreMesh(core_axis_name="c", subcore_axis_name="s")

    def body(tok_hbm, big_hbm, out_hbm, idx_v, *bufs_and_sems):
        rows = bufs_and_sems[:_NBUF]
        sem_g = bufs_and_sems[_NBUF:2 * _NBUF]
        sem_w = bufs_and_sems[2 * _NBUF:3 * _NBUF]
        wid = lax.axis_index("s") * nc + lax.axis_index("c")
        w_base = wid * per_w
        pltpu.sync_copy(tok_hbm.at[pl.ds(w_base, per_w)], idx_v)

        def out_slice(g, b):
            base = pl.multiple_of(
                w_base + (g * _NBUF + b) * _CHUNK, _CHUNK)
            return out_hbm.at[pl.ds(base, _CHUNK)]

        def idx_slice(g, b):
            off = pl.multiple_of((g * _NBUF + b) * _CHUNK, _CHUNK)
            return idx_v.at[pl.ds(off, _CHUNK)]

        def group(g, carry):
            gathers = []
            for b in range(_NBUF):
                @pl.when(g > 0)
                def _(b=b):
                    pltpu.make_async_copy(
                        rows[b], out_slice(g - 1, b), sem_w[b]).wait()
                gathers.append(pltpu.async_copy(
                    big_hbm.at[idx_slice(g, b)], rows[b], sem_g[b]))
            for b in range(_NBUF):
                gathers[b].wait()
                pltpu.async_copy(rows[b], out_slice(g, b), sem_w[b])
            return carry

        lax.fori_loop(0, n_groups, group, 0)
        for b in range(_NBUF):
            pltpu.make_async_copy(
                rows[b], out_slice(n_groups - 1, b), sem_w[b]).wait()

    return pl.kernel(
        body,
        out_type=jax.ShapeDtypeStruct((n_tokens, D_OUT), jnp.float32),
        mesh=mesh,
        scratch_types=(
            [pltpu.VMEM((n_tokens // nw,), jnp.int32)]
            + [pltpu.VMEM((_CHUNK, D_OUT), jnp.float32)] * _NBUF
            + [pltpu.SemaphoreType.DMA] * (2 * _NBUF)
        ),
    )


def _gate_mats(gate_w, gate_b):
    eye = jnp.eye(_GPACK, dtype=jnp.float32)
    w8 = jnp.einsum("ml,ji->mjli", eye, gate_w).reshape(
        _GPACK * D_BIN, _GPACK * D_BIN)
    b8 = jnp.tile(gate_b, _GPACK).reshape(_GPACK * D_BIN, 1)
    return w8, b8


def kernel(tokens, binary_table, analog_table, fractal_table, gate_w, gate_b):
    batch, hist = tokens.shape
    n = batch * hist
    v = binary_table.shape[0]
    tok = tokens.T.reshape(n).astype(jnp.int32)
    w8, b8 = _gate_mats(gate_w, gate_b)
    big = _pack_tables(binary_table.T, analog_table.T, fractal_table.T, w8, b8)
    out = _sc_gather_fn(n, v)(tok, big)
    return out.reshape(hist, batch, D_OUT).transpose(1, 0, 2)
